# Initial kernel scaffold; baseline (speedup 1.0000x reference)
import jax
import jax.numpy as jnp
from jax.experimental import pallas as pl


def kernel(input):
    raise NotImplementedError("write your pallas kernel here")


# speedup vs baseline: 65.5171x; 65.5171x over previous
import functools

import numpy as np
import jax
import jax.numpy as jnp
from jax import lax
from jax.experimental import pallas as pl
from jax.experimental.pallas import tpu as pltpu
from jax.experimental.pallas import tpu_sc as plsc

NCLS = 10
NC = 2
NS = 16
NW = NC * NS
LANES = 16
UNROLL = 8
HBINS = 2048
PASSES = ((21, 11), (10, 11), (0, 10))
SIGNBIT = np.int32(-(1 << 31))


def _draw_terms(H):
    rng = np.random.default_rng(0)
    terms = []
    for _ in range(1, NCLS):
        _ = int(rng.integers(1, min(5, H) + 1))
        ts = int(rng.integers(1, min(3, H) + 1))
        sel = [int(v) for v in rng.integers(0, H, size=(ts,))]
        sg = [bool(v) for v in rng.integers(0, 2, size=(ts,))]
        terms.append((sel, sg))
    return terms


def _skey(bits):
    neg = bits ^ jnp.int32(0x7FFFFFFF)
    return jnp.where(bits >= 0, bits,
                     jnp.where(bits == SIGNBIT, jnp.int32(0), neg))


def _med_body(T, k, xs_hbm, med_out, colbuf, hist, medbuf):
    wid = lax.axis_index("s") * NC + lax.axis_index("c")
    pltpu.sync_copy(xs_hbm.at[wid], colbuf)
    nchunks = T // LANES
    ones = jnp.ones((LANES,), jnp.int32)
    lanes = lax.iota(jnp.int32, LANES)
    laneoff = lanes * jnp.int32(HBINS)

    def xform(i, _):
        for u in range(UNROLL):
            idx = pl.ds((i * UNROLL + u) * LANES, LANES)
            colbuf[idx] = _skey(colbuf[idx])
        return 0

    lax.fori_loop(0, nchunks // UNROLL, xform, 0)

    P = jnp.int32(0)
    kk = jnp.int32(k)
    bits_done = 0
    for shift, nb in PASSES:
        nbins = 1 << nb

        def zero(i, _):
            for u in range(UNROLL):
                idx = pl.ds((i * UNROLL + u) * LANES, LANES)
                hist[idx] = jnp.zeros((LANES,), jnp.int32)
            return 0

        lax.fori_loop(0, (LANES * HBINS) // (LANES * UNROLL), zero, 0)

        flip_d = jnp.int32(1 << (31 - shift)) if shift > 0 else SIGNBIT
        mask_d = jnp.int32(nbins - 1)
        if bits_done:
            flip_m = jnp.int32(1 << (bits_done - 1))
            Pcur = P

        def accum(i, _):
            for u in range(UNROLL):
                idx = pl.ds((i * UNROLL + u) * LANES, LANES)
                sk = colbuf[idx]
                digit = (lax.shift_right_logical(sk, shift) ^ flip_d) & mask_d
                if bits_done:
                    m = (lax.shift_right_logical(sk, shift + nb) ^ flip_m) == Pcur
                    plsc.addupdate_scatter(hist, [laneoff + digit], ones, mask=m)
                else:
                    plsc.addupdate_scatter(hist, [laneoff + digit], ones)
            return 0

        lax.fori_loop(0, nchunks // UNROLL, accum, 0)

        def scan(i, carry):
            run, bcnt, cmx = carry
            h = hist[pl.ds(i * LANES, LANES)]
            for l in range(1, LANES):
                h = h + hist[pl.ds(l * HBINS + i * LANES, LANES)]
            cum = run + plsc.cumsum(h)
            le = cum <= kk
            bcnt = bcnt + jnp.sum(jnp.where(le, 1, 0).astype(jnp.int32))
            cmx = jnp.maximum(cmx, jnp.max(jnp.where(le, cum, jnp.int32(0))))
            return jnp.max(cum), bcnt, cmx

        z = jnp.int32(0)
        _, b, cmx = lax.fori_loop(0, nbins // LANES, scan, (z, z, z))
        kk = kk - cmx
        P = (P << nb) | b
        bits_done += nb

    medk = P ^ SIGNBIT
    medbuf[...] = jnp.zeros((LANES,), jnp.int32) + medk
    pltpu.sync_copy(medbuf, med_out.at[pl.ds(wid * LANES, LANES)])


def _bits_body(T, terms, slots, d, xs_hbm, med_hbm, mask_out, cnt_out,
               rowbuf, medv, maskbuf, cntrow):
    wid = lax.axis_index("s") * NC + lax.axis_index("c")
    RB = T // NW
    base = wid * RB
    for j in range(d):
        pltpu.sync_copy(xs_hbm.at[j, pl.ds(base, RB)],
                        rowbuf.at[pl.ds(j * RB, RB)])
    pltpu.sync_copy(med_hbm, medv)
    meds = [medv[pl.ds(j * LANES, LANES)] for j in range(d)]
    lanes = lax.iota(jnp.int32, LANES)

    def step(i, cnt):
        off = i * LANES
        bits = []
        for j in range(d):
            bits.append(_skey(rowbuf[pl.ds(j * RB + off, LANES)]) > meds[j])
        bmask = jnp.zeros((LANES,), jnp.int32)
        for ci, (sel, sg) in enumerate(terms):
            c = ci + 1
            m = None
            for s, g in zip(sel, sg):
                t = bits[slots[s]] if g else jnp.logical_not(bits[slots[s]])
                m = t if m is None else jnp.logical_and(m, t)
            bmask = bmask + jnp.where(m, jnp.int32(1 << c), jnp.int32(0))
            pc = jnp.sum(jnp.where(m, 1, 0).astype(jnp.int32))
            cnt = cnt + jnp.where(lanes == c, pc, jnp.int32(0))
        maskbuf[pl.ds(off, LANES)] = bmask
        return cnt

    cnt = lax.fori_loop(0, RB // LANES, step, jnp.zeros((LANES,), jnp.int32))
    cntrow[...] = cnt
    pltpu.sync_copy(cntrow, cnt_out.at[pl.ds(wid * LANES, LANES)])
    pltpu.sync_copy(maskbuf, mask_out.at[pl.ds(base, RB)])


def _label_body(T, thresh, mask_hbm, cnt_hbm, out_hbm,
                maskbuf, cntv, lblbuf):
    wid = lax.axis_index("s") * NC + lax.axis_index("c")
    RB = T // NW
    base = wid * RB
    pltpu.sync_copy(cnt_hbm, cntv)
    tot = jnp.zeros((LANES,), jnp.int32)
    for i in range(NW):
        tot = tot + cntv[pl.ds(i * LANES, LANES)]
    lanes = lax.iota(jnp.int32, LANES)
    inrange = jnp.logical_and(lanes >= 1, lanes <= NCLS - 1)
    exceed = jnp.where(jnp.logical_and(tot > thresh, inrange), 1, 0)
    exceed = exceed.astype(jnp.int32)
    cs = plsc.cumsum(exceed)
    active = jnp.logical_and((cs - exceed) == 0, inrange)
    bitvals = lax.shift_left(jnp.int32(1), lanes)
    actmask = jnp.sum(jnp.where(active, bitvals, jnp.int32(0)))
    pltpu.sync_copy(mask_hbm.at[pl.ds(base, RB)], maskbuf)

    def step(i, _):
        idx = pl.ds(i * LANES, LANES)
        m = maskbuf[idx] & actmask
        lbl = jnp.zeros((LANES,), jnp.int32)
        for c in range(1, NCLS):
            hit = (lax.shift_right_logical(m, c) & 1) == 1
            lbl = jnp.where(hit, jnp.int32(c), lbl)
        lblbuf[idx] = lbl
        return 0

    lax.fori_loop(0, RB // LANES, step, 0)
    pltpu.sync_copy(lblbuf, out_hbm.at[pl.ds(base, RB)])


def kernel(input):
    x = input
    if x.ndim == 1:
        x = x[:, None]
    T, H = x.shape
    terms = _draw_terms(H)
    cols = sorted({s for sel, _ in terms for s in sel})
    slots = {c: i for i, c in enumerate(cols)}
    d = len(cols)
    cols_pad = cols + [cols[0]] * (NW - d)
    k = (T - 1) // 2
    thresh = T // (2 * NCLS)
    RB = T // NW

    xs_t = jnp.take(x.astype(jnp.float32),
                    jnp.asarray(cols_pad, dtype=jnp.int32), axis=1).T
    xs_t = lax.bitcast_convert_type(xs_t, jnp.int32)

    mesh = plsc.VectorSubcoreMesh(core_axis_name="c", subcore_axis_name="s",
                                  num_cores=NC, num_subcores=NS)
    cparams = pltpu.CompilerParams(needs_layout_passes=False)

    med = pl.kernel(
        functools.partial(_med_body, T, k),
        out_type=jax.ShapeDtypeStruct((NW * LANES,), jnp.int32),
        mesh=mesh,
        compiler_params=cparams,
        scratch_types=[
            pltpu.VMEM((T,), jnp.int32),
            pltpu.VMEM((LANES * HBINS,), jnp.int32),
            pltpu.VMEM((LANES,), jnp.int32),
        ],
    )(xs_t)

    bmask, cnts = pl.kernel(
        functools.partial(_bits_body, T, terms, slots, d),
        out_type=[jax.ShapeDtypeStruct((T,), jnp.int32),
                  jax.ShapeDtypeStruct((NW * LANES,), jnp.int32)],
        mesh=mesh,
        compiler_params=cparams,
        scratch_types=[
            pltpu.VMEM((d * RB,), jnp.int32),
            pltpu.VMEM((NW * LANES,), jnp.int32),
            pltpu.VMEM((RB,), jnp.int32),
            pltpu.VMEM((LANES,), jnp.int32),
        ],
    )(xs_t, med)

    out = pl.kernel(
        functools.partial(_label_body, T, thresh),
        out_type=jax.ShapeDtypeStruct((T,), jnp.int32),
        mesh=mesh,
        compiler_params=cparams,
        scratch_types=[
            pltpu.VMEM((RB,), jnp.int32),
            pltpu.VMEM((NW * LANES,), jnp.int32),
            pltpu.VMEM((RB,), jnp.int32),
        ],
    )(bmask, cnts)

    return out
